# baseline (speedup 1.0000x reference)
import functools
import math

import jax
import jax.numpy as jnp
from jax.experimental import pallas as pl

T = 2.0
LOG_2PI = math.log(2.0 * math.pi)


def _flow_kernel(theta_ref, ctx_ref, wmu_ref, bmu_ref, wls_ref, bls_ref, out_ref):
    ctx = ctx_ref[...]
    theta = theta_ref[...]
    mu = jnp.dot(ctx, wmu_ref[...], preferred_element_type=jnp.float32) + bmu_ref[...]
    ls = jnp.tanh(
        jnp.dot(ctx, wls_ref[...], preferred_element_type=jnp.float32) + bls_ref[...]
    )
    z = (theta - mu) * jnp.exp(-ls)
    d = theta.shape[-1]
    red = jnp.sum(z * z + 2.0 * ls, axis=-1)
    out_ref[...] = (-0.5 / T) * red - (0.5 * d * LOG_2PI / T)


@jax.jit
def kernel(theta, context, W_mu, b_mu, W_ls, b_ls):
    n, d = theta.shape
    c = context.shape[-1]
    tile = 2048
    grid = (n // tile,)
    out = pl.pallas_call(
        _flow_kernel,
        grid=grid,
        in_specs=[
            pl.BlockSpec((tile, d), lambda i: (i, 0)),
            pl.BlockSpec((tile, c), lambda i: (i, 0)),
            pl.BlockSpec((c, d), lambda i: (0, 0)),
            pl.BlockSpec((d,), lambda i: (0,)),
            pl.BlockSpec((c, d), lambda i: (0, 0)),
            pl.BlockSpec((d,), lambda i: (0,)),
        ],
        out_specs=pl.BlockSpec((tile,), lambda i: (i,)),
        out_shape=jax.ShapeDtypeStruct((n,), jnp.float32),
    )(theta, context, W_mu, b_mu, W_ls, b_ls)
    return out


# speedup vs baseline: 1.1287x; 1.1287x over previous
import math

import jax
import jax.numpy as jnp
from jax.experimental import pallas as pl

T = 2.0
LOG_2PI = math.log(2.0 * math.pi)


def _flow_kernel(theta_ref, ctx_ref, wmu_ref, bmu_ref, wls_ref, bls_ref, out_ref):
    ctx = ctx_ref[...]
    theta = theta_ref[...]
    mu = jnp.dot(ctx, wmu_ref[...], preferred_element_type=jnp.float32) + bmu_ref[...]
    ls = jnp.tanh(
        jnp.dot(ctx, wls_ref[...], preferred_element_type=jnp.float32) + bls_ref[...]
    )
    z = (theta - mu) * jnp.exp(-ls)
    d = theta.shape[-1]
    v = z * z + 2.0 * ls
    w_red = jnp.full((d, 1), -0.5 / T, dtype=jnp.float32)
    out_ref[...] = jnp.dot(v, w_red, preferred_element_type=jnp.float32) - (
        0.5 * d * LOG_2PI / T
    )


@jax.jit
def kernel(theta, context, W_mu, b_mu, W_ls, b_ls):
    n, d = theta.shape
    c = context.shape[-1]
    tile = 2048
    grid = (n // tile,)
    out = pl.pallas_call(
        _flow_kernel,
        grid=grid,
        in_specs=[
            pl.BlockSpec((tile, d), lambda i: (i, 0)),
            pl.BlockSpec((tile, c), lambda i: (i, 0)),
            pl.BlockSpec((c, d), lambda i: (0, 0)),
            pl.BlockSpec((d,), lambda i: (0,)),
            pl.BlockSpec((c, d), lambda i: (0, 0)),
            pl.BlockSpec((d,), lambda i: (0,)),
        ],
        out_specs=pl.BlockSpec((tile, 1), lambda i: (i, 0)),
        out_shape=jax.ShapeDtypeStruct((n, 1), jnp.float32),
    )(theta, context, W_mu, b_mu, W_ls, b_ls)
    return out[:, 0]


# speedup vs baseline: 3.3081x; 2.9310x over previous
import math

import jax
import jax.numpy as jnp
from jax import lax
from jax.experimental import pallas as pl

T = 2.0
LOG_2PI = math.log(2.0 * math.pi)
_CONTRACT = (((1,), (1,)), ((), ()))


def _flow_kernel(thetaT_ref, ctx_ref, wmuT_ref, bmu_ref, wlsT_ref, bls_ref, out_ref):
    ctx = ctx_ref[...]
    mt = lax.dot_general(
        wmuT_ref[...], ctx, _CONTRACT, preferred_element_type=jnp.float32
    )
    lt = lax.dot_general(
        wlsT_ref[...], ctx, _CONTRACT, preferred_element_type=jnp.float32
    )
    mu = mt + bmu_ref[...][:, None]
    ls = jnp.tanh(lt + bls_ref[...][:, None])
    z = (thetaT_ref[...] - mu) * jnp.exp(-ls)
    v = z * z + 2.0 * ls
    d = v.shape[0]
    out_ref[...] = (-0.5 / T) * jnp.sum(v, axis=0) - (0.5 * d * LOG_2PI / T)


@jax.jit
def kernel(theta, context, W_mu, b_mu, W_ls, b_ls):
    n, d = theta.shape
    c = context.shape[-1]
    tile = 2048
    grid = (n // tile,)
    return pl.pallas_call(
        _flow_kernel,
        grid=grid,
        in_specs=[
            pl.BlockSpec((d, tile), lambda i: (0, i)),
            pl.BlockSpec((tile, c), lambda i: (i, 0)),
            pl.BlockSpec((d, c), lambda i: (0, 0)),
            pl.BlockSpec((d,), lambda i: (0,)),
            pl.BlockSpec((d, c), lambda i: (0, 0)),
            pl.BlockSpec((d,), lambda i: (0,)),
        ],
        out_specs=pl.BlockSpec((tile,), lambda i: (i,)),
        out_shape=jax.ShapeDtypeStruct((n,), jnp.float32),
    )(theta.T, context, W_mu.T, b_mu, W_ls.T, b_ls)


# speedup vs baseline: 3.8647x; 1.1683x over previous
import math

import jax
import jax.numpy as jnp
from jax import lax
from jax.experimental import pallas as pl
from jax.experimental.pallas import tpu as pltpu

T = 2.0
LOG_2PI = math.log(2.0 * math.pi)
_CONTRACT = (((1,), (1,)), ((), ()))

_TILE = 2048
_DEPTH = 4


def _flow_kernel(
    thetaT_hbm,
    ctx_hbm,
    wmuT_ref,
    bmu_ref,
    wlsT_ref,
    bls_ref,
    out_ref,
    th_buf,
    ctx_buf,
    sems,
):
    n = out_ref.shape[0]
    nchunk = n // _TILE

    def th_copy(c, slot):
        return pltpu.make_async_copy(
            thetaT_hbm.at[:, pl.ds(c * _TILE, _TILE)], th_buf.at[slot], sems.at[0, slot]
        )

    def ctx_copy(c, slot):
        return pltpu.make_async_copy(
            ctx_hbm.at[pl.ds(c * _TILE, _TILE), :], ctx_buf.at[slot], sems.at[1, slot]
        )

    for k in range(_DEPTH - 1):
        th_copy(k, k).start()
        ctx_copy(k, k).start()

    wmuT = wmuT_ref[...]
    wlsT = wlsT_ref[...]
    bmu = bmu_ref[...][:, None]
    bls = bls_ref[...][:, None]
    d = wmuT.shape[0]
    const = 0.5 * d * LOG_2PI / T

    for i in range(nchunk):
        slot = i % _DEPTH
        th_copy(i, slot).wait()
        ctx_copy(i, slot).wait()
        nxt = i + _DEPTH - 1
        if nxt < nchunk:
            th_copy(nxt, nxt % _DEPTH).start()
            ctx_copy(nxt, nxt % _DEPTH).start()
        ctx = ctx_buf[slot]
        mt = lax.dot_general(wmuT, ctx, _CONTRACT, preferred_element_type=jnp.float32)
        lt = lax.dot_general(wlsT, ctx, _CONTRACT, preferred_element_type=jnp.float32)
        mu = mt + bmu
        ls = jnp.tanh(lt + bls)
        z = (th_buf[slot] - mu) * jnp.exp(-ls)
        v = z * z + 2.0 * ls
        out_ref[pl.ds(i * _TILE, _TILE)] = (-0.5 / T) * jnp.sum(v, axis=0) - const


@jax.jit
def kernel(theta, context, W_mu, b_mu, W_ls, b_ls):
    n, d = theta.shape
    c = context.shape[-1]
    return pl.pallas_call(
        _flow_kernel,
        in_specs=[
            pl.BlockSpec(memory_space=pl.ANY),
            pl.BlockSpec(memory_space=pl.ANY),
            pl.BlockSpec((d, c), lambda: (0, 0)),
            pl.BlockSpec((d,), lambda: (0,)),
            pl.BlockSpec((d, c), lambda: (0, 0)),
            pl.BlockSpec((d,), lambda: (0,)),
        ],
        out_specs=pl.BlockSpec((n,), lambda: (0,)),
        out_shape=jax.ShapeDtypeStruct((n,), jnp.float32),
        scratch_shapes=[
            pltpu.VMEM((_DEPTH, d, _TILE), jnp.float32),
            pltpu.VMEM((_DEPTH, _TILE, c), jnp.float32),
            pltpu.SemaphoreType.DMA((2, _DEPTH)),
        ],
    )(theta.T, context, W_mu.T, b_mu, W_ls.T, b_ls)
